# baseline (speedup 1.0000x reference)
import functools

import jax
import jax.numpy as jnp
from jax import lax
from jax.experimental import pallas as pl
from jax.experimental.pallas import tpu as pltpu
from jax.experimental.pallas import tpu_sc as plsc

_NATTRS = 115
_NOBJS = 245
_NNODES = _NATTRS + _NOBJS
_NEDGES = 57000

_NC = 2
_NS = 16
_NW = _NC * _NS
_CHUNK = 128
_NCHUNKS = 14
_EPW = _CHUNK * _NCHUNKS
_NE_PAD = _NW * _EPW
_AVALID = _NNODES * _NNODES
_ASIZE = 131072
_SLICE = _ASIZE // _NS

_PBLK = 2048
_NPAIRS = _NATTRS * _NOBJS
_PGRID = (_NPAIRS + _PBLK - 1) // _PBLK


def _sc_adj_body(src_hbm, dst_hbm, out_hbm, src_v, dst_v, idx_v, ones_v,
                 stage_v, shared):
    c = lax.axis_index("c")
    s = lax.axis_index("s")
    wid = s * _NC + c
    base = wid * _EPW

    def _z(i, carry):
        stage_v[pl.ds(i * 16, 16)] = jnp.zeros((16,), jnp.float32)
        return carry

    lax.fori_loop(0, _SLICE // 16, _z, 0)
    pltpu.sync_copy(stage_v, shared.at[pl.ds(s * _SLICE, _SLICE)])

    pltpu.sync_copy(src_hbm.at[pl.ds(base, _EPW)], src_v)
    pltpu.sync_copy(dst_hbm.at[pl.ds(base, _EPW)], dst_v)
    for t in range(_CHUNK // 16):
        ones_v[pl.ds(t * 16, 16)] = jnp.ones((16,), jnp.float32)

    for k in range(_NCHUNKS):
        for t in range(_CHUNK // 16):
            o = k * _CHUNK + t * 16
            d16 = dst_v[pl.ds(o, 16)]
            s16 = src_v[pl.ds(o, 16)]
            idx_v[k, pl.ds(t * 16, 16)] = d16 * _NNODES + s16

    plsc.subcore_barrier()
    for k in range(_NCHUNKS):
        pltpu.sync_copy(ones_v, shared.at[idx_v.at[k]], add=True)
    plsc.subcore_barrier()

    pltpu.sync_copy(shared.at[pl.ds(s * _SLICE, _SLICE)], stage_v)
    pltpu.sync_copy(stage_v, out_hbm.at[c, pl.ds(s * _SLICE, _SLICE)])


def _build_adjacency(src_p, dst_p):
    return pl.kernel(
        _sc_adj_body,
        out_type=jax.ShapeDtypeStruct((_NC, _ASIZE), jnp.float32),
        mesh=plsc.VectorSubcoreMesh(
            core_axis_name="c", subcore_axis_name="s",
            num_cores=_NC, num_subcores=_NS),
        scratch_types=[
            pltpu.VMEM((_EPW,), jnp.int32),
            pltpu.VMEM((_EPW,), jnp.int32),
            pltpu.VMEM((_NCHUNKS, _CHUNK), jnp.int32),
            pltpu.VMEM((_CHUNK,), jnp.float32),
            pltpu.VMEM((_SLICE,), jnp.float32),
            pltpu.VMEM_SHARED((_ASIZE,), jnp.float32),
        ],
    )(src_p, dst_p)


def _ln(x, g, b):
    m = jnp.mean(x, axis=-1, keepdims=True)
    v = jnp.mean((x - m) * (x - m), axis=-1, keepdims=True)
    return (x - m) / jnp.sqrt(v + 1e-5) * g + b


def _dot(a, b):
    return jnp.dot(a, b, preferred_element_type=jnp.float32)


def _gnn_body(a_ref, nodes_ref, wr1, wn1, b1, wr2, wn2, b2, h2_out):
    A = a_ref[0] + a_ref[1]
    deg = jnp.sum(A, axis=1, keepdims=True)
    An = A / jnp.maximum(deg, 1.0)
    x = nodes_ref[...]
    agg1 = _dot(An, x)
    h1 = jnp.maximum(_dot(x, wr1[...]) + _dot(agg1, wn1[...]) + b1[...], 0.0)
    agg2 = _dot(An, h1)
    h2_out[...] = _dot(h1, wr2[...]) + _dot(agg2, wn2[...]) + b2[...]


def _img_body(img_ref, w1, b1, g1, be1, w2, b2, g2, be2, w3, b3, go, beo,
              f_out):
    x = img_ref[...]
    x = jnp.maximum(_ln(_dot(x, w1[...]) + b1[...], g1[...], be1[...]), 0.0)
    x = jnp.maximum(_ln(_dot(x, w2[...]) + b2[...], g2[...], be2[...]), 0.0)
    f_out[...] = _ln(_dot(x, w3[...]) + b3[...], go[...], beo[...])


def _pair_body(attr_ref, obj_ref, wa, wb, b1, g1, be1, w2, b2, go, beo,
               f_ref, out_ref, ap_s, op_s):
    i = pl.program_id(0)

    @pl.when(i == 0)
    def _():
        ap_s[...] = _dot(attr_ref[...], wa[...])
        op_s[...] = _dot(obj_ref[...], wb[...])

    r = i * _PBLK + lax.broadcasted_iota(jnp.int32, (_PBLK, 1), 0)
    ia = r // _NOBJS
    jo = r - ia * _NOBJS
    aoh = (ia == lax.broadcasted_iota(jnp.int32, (_PBLK, 128), 1)
           ).astype(jnp.bfloat16)
    boh = (jo == lax.broadcasted_iota(jnp.int32, (_PBLK, 256), 1)
           ).astype(jnp.bfloat16)
    x = (_dot(aoh, ap_s[...].astype(jnp.bfloat16))
         + _dot(boh, op_s[...].astype(jnp.bfloat16)) + b1[...])
    x = jnp.maximum(_ln(x, g1[...], be1[...]), 0.0)
    q = _dot(x.astype(jnp.bfloat16), w2[...].astype(jnp.bfloat16)) + b2[...]
    p = _ln(q, go[...], beo[...])
    out_ref[...] = lax.dot_general(
        f_ref[...].astype(jnp.bfloat16), p.astype(jnp.bfloat16),
        (((1,), (1,)), ((), ())), preferred_element_type=jnp.float32)


def _full(i):
    return (0, 0)


def kernel(img, edge_index, nodes, sage1_wr, sage1_wn, sage1_b, sage2_wr,
           sage2_wn, sage2_b, img_w1, img_b1, img_g1, img_be1, img_w2,
           img_b2, img_g2, img_be2, img_w3, img_b3, img_go, img_beo,
           pair_w1, pair_b1, pair_g1, pair_be1, pair_w2, pair_b2, pair_go,
           pair_beo):
    src = edge_index[0].astype(jnp.int32)
    dst = edge_index[1].astype(jnp.int32)
    pad = _NE_PAD - _NEDGES
    src_p = jnp.concatenate([src, jnp.zeros((pad,), jnp.int32)])
    dst_p = jnp.concatenate([dst, jnp.full((pad,), _NNODES, jnp.int32)])

    a2 = _build_adjacency(src_p, dst_p)
    a = a2[:, :_AVALID].reshape(_NC, _NNODES, _NNODES)

    h2 = pl.pallas_call(
        _gnn_body,
        out_shape=jax.ShapeDtypeStruct((_NNODES, 512), jnp.float32),
    )(a, nodes, sage1_wr, sage1_wn, sage1_b.reshape(1, -1),
      sage2_wr, sage2_wn, sage2_b.reshape(1, -1))

    f = pl.pallas_call(
        _img_body,
        out_shape=jax.ShapeDtypeStruct((img.shape[0], 800), jnp.float32),
    )(img, img_w1, img_b1.reshape(1, -1), img_g1.reshape(1, -1),
      img_be1.reshape(1, -1), img_w2, img_b2.reshape(1, -1),
      img_g2.reshape(1, -1), img_be2.reshape(1, -1), img_w3,
      img_b3.reshape(1, -1), img_go.reshape(1, -1), img_beo.reshape(1, -1))

    attr_pad = jnp.zeros((128, 512), jnp.float32).at[:_NATTRS].set(
        h2[:_NATTRS])
    obj_pad = jnp.zeros((256, 512), jnp.float32).at[:_NOBJS].set(
        h2[_NATTRS:])
    wa = pair_w1[:512]
    wb = pair_w1[512:]

    out = pl.pallas_call(
        _pair_body,
        grid=(_PGRID,),
        in_specs=[
            pl.BlockSpec((128, 512), _full),
            pl.BlockSpec((256, 512), _full),
            pl.BlockSpec((512, 1200), _full),
            pl.BlockSpec((512, 1200), _full),
            pl.BlockSpec((1, 1200), _full),
            pl.BlockSpec((1, 1200), _full),
            pl.BlockSpec((1, 1200), _full),
            pl.BlockSpec((1200, 800), _full),
            pl.BlockSpec((1, 800), _full),
            pl.BlockSpec((1, 800), _full),
            pl.BlockSpec((1, 800), _full),
            pl.BlockSpec((1024, 800), _full),
        ],
        out_specs=pl.BlockSpec((1024, _PBLK), lambda i: (0, i)),
        out_shape=jax.ShapeDtypeStruct((1024, _NPAIRS), jnp.float32),
        scratch_shapes=[
            pltpu.VMEM((128, 1200), jnp.float32),
            pltpu.VMEM((256, 1200), jnp.float32),
        ],
    )(attr_pad, obj_pad, wa, wb, pair_b1.reshape(1, -1),
      pair_g1.reshape(1, -1), pair_be1.reshape(1, -1), pair_w2,
      pair_b2.reshape(1, -1), pair_go.reshape(1, -1),
      pair_beo.reshape(1, -1), f)
    return out


# speedup vs baseline: 1.3000x; 1.3000x over previous
import functools

import jax
import jax.numpy as jnp
from jax import lax
from jax.experimental import pallas as pl
from jax.experimental.pallas import tpu as pltpu
from jax.experimental.pallas import tpu_sc as plsc

_NATTRS = 115
_NOBJS = 245
_NNODES = _NATTRS + _NOBJS
_NEDGES = 57000

_NC = 2
_NS = 16
_NW = _NC * _NS
_CHUNK = 128
_NCHUNKS = 14
_EPW = _CHUNK * _NCHUNKS
_NE_PAD = _NW * _EPW
_AVALID = _NNODES * _NNODES
_ASIZE = 131072
_SLICE = _ASIZE // _NS

_PBLK = 2048
_NPAIRS = _NATTRS * _NOBJS
_PGRID = (_NPAIRS + _PBLK - 1) // _PBLK


def _sc_adj_body(src_hbm, dst_hbm, out_hbm, src_v, dst_v, idx_v, ones_v,
                 stage_v, shared):
    c = lax.axis_index("c")
    s = lax.axis_index("s")
    wid = s * _NC + c
    base = wid * _EPW

    def _z(i, carry):
        stage_v[pl.ds(i * 16, 16)] = jnp.zeros((16,), jnp.float32)
        return carry

    lax.fori_loop(0, _SLICE // 16, _z, 0)
    pltpu.sync_copy(stage_v, shared.at[pl.ds(s * _SLICE, _SLICE)])

    pltpu.sync_copy(src_hbm.at[pl.ds(base, _EPW)], src_v)
    pltpu.sync_copy(dst_hbm.at[pl.ds(base, _EPW)], dst_v)
    for t in range(_CHUNK // 16):
        ones_v[pl.ds(t * 16, 16)] = jnp.ones((16,), jnp.float32)

    for k in range(_NCHUNKS):
        for t in range(_CHUNK // 16):
            o = k * _CHUNK + t * 16
            d16 = dst_v[pl.ds(o, 16)]
            s16 = src_v[pl.ds(o, 16)]
            idx_v[k, pl.ds(t * 16, 16)] = d16 * _NNODES + s16

    plsc.subcore_barrier()
    for k in range(_NCHUNKS):
        pltpu.sync_copy(ones_v, shared.at[idx_v.at[k]], add=True)
    plsc.subcore_barrier()

    pltpu.sync_copy(shared.at[pl.ds(s * _SLICE, _SLICE)], stage_v)
    pltpu.sync_copy(stage_v, out_hbm.at[c, pl.ds(s * _SLICE, _SLICE)])


def _build_adjacency(src_p, dst_p):
    return pl.kernel(
        _sc_adj_body,
        out_type=jax.ShapeDtypeStruct((_NC, _ASIZE), jnp.float32),
        mesh=plsc.VectorSubcoreMesh(
            core_axis_name="c", subcore_axis_name="s",
            num_cores=_NC, num_subcores=_NS),
        scratch_types=[
            pltpu.VMEM((_EPW,), jnp.int32),
            pltpu.VMEM((_EPW,), jnp.int32),
            pltpu.VMEM((_NCHUNKS, _CHUNK), jnp.int32),
            pltpu.VMEM((_CHUNK,), jnp.float32),
            pltpu.VMEM((_SLICE,), jnp.float32),
            pltpu.VMEM_SHARED((_ASIZE,), jnp.float32),
        ],
    )(src_p, dst_p)


def _ln(x, g, b):
    m = jnp.mean(x, axis=-1, keepdims=True)
    v = jnp.mean((x - m) * (x - m), axis=-1, keepdims=True)
    return (x - m) / jnp.sqrt(v + 1e-5) * g + b


def _dot(a, b):
    return jnp.dot(a, b, preferred_element_type=jnp.float32)


def _gnn_body(a_ref, nodes_ref, wr1, wn1, b1, wr2, wn2, b2, h2_out):
    A = a_ref[0] + a_ref[1]
    deg = jnp.sum(A, axis=1, keepdims=True)
    An = A / jnp.maximum(deg, 1.0)
    x = nodes_ref[...]
    agg1 = _dot(An, x)
    h1 = jnp.maximum(_dot(x, wr1[...]) + _dot(agg1, wn1[...]) + b1[...], 0.0)
    agg2 = _dot(An, h1)
    h2_out[...] = _dot(h1, wr2[...]) + _dot(agg2, wn2[...]) + b2[...]


def _img_body(img_ref, w1, b1, g1, be1, w2, b2, g2, be2, w3, b3, go, beo,
              f_out):
    x = img_ref[...]
    x = jnp.maximum(_ln(_dot(x, w1[...]) + b1[...], g1[...], be1[...]), 0.0)
    x = jnp.maximum(_ln(_dot(x, w2[...]) + b2[...], g2[...], be2[...]), 0.0)
    f_out[...] = _ln(_dot(x, w3[...]) + b3[...], go[...], beo[...])


def _pair_body(attr_ref, obj_ref, wa, wb, b1, g1, be1, w2, b2, go, beo,
               f_ref, out_ref, ap_s, op_s):
    i = pl.program_id(0)

    @pl.when(i == 0)
    def _():
        ap_s[...] = _dot(attr_ref[...], wa[...])
        op_s[...] = _dot(obj_ref[...], wb[...])

    r = i * _PBLK + lax.broadcasted_iota(jnp.int32, (_PBLK, 1), 0)
    ia = r // _NOBJS
    jo = r - ia * _NOBJS
    aoh = (ia == lax.broadcasted_iota(jnp.int32, (_PBLK, 128), 1)
           ).astype(jnp.float32)
    boh = (jo == lax.broadcasted_iota(jnp.int32, (_PBLK, 256), 1)
           ).astype(jnp.float32)
    x = _dot(aoh, ap_s[...]) + _dot(boh, op_s[...]) + b1[...]
    x = jnp.maximum(_ln(x, g1[...], be1[...]), 0.0)
    q = _dot(x, w2[...]) + b2[...]
    p = _ln(q, go[...], beo[...])
    out_ref[...] = lax.dot_general(
        p, f_ref[...], (((1,), (1,)), ((), ())),
        preferred_element_type=jnp.float32)


def _full(i):
    return (0, 0)


def kernel(img, edge_index, nodes, sage1_wr, sage1_wn, sage1_b, sage2_wr,
           sage2_wn, sage2_b, img_w1, img_b1, img_g1, img_be1, img_w2,
           img_b2, img_g2, img_be2, img_w3, img_b3, img_go, img_beo,
           pair_w1, pair_b1, pair_g1, pair_be1, pair_w2, pair_b2, pair_go,
           pair_beo):
    src = edge_index[0].astype(jnp.int32)
    dst = edge_index[1].astype(jnp.int32)
    pad = _NE_PAD - _NEDGES
    src_p = jnp.concatenate([src, jnp.zeros((pad,), jnp.int32)])
    dst_p = jnp.concatenate([dst, jnp.full((pad,), _NNODES, jnp.int32)])

    a2 = _build_adjacency(src_p, dst_p)
    a = a2[:, :_AVALID].reshape(_NC, _NNODES, _NNODES)

    h2 = pl.pallas_call(
        _gnn_body,
        out_shape=jax.ShapeDtypeStruct((_NNODES, 512), jnp.float32),
    )(a, nodes, sage1_wr, sage1_wn, sage1_b.reshape(1, -1),
      sage2_wr, sage2_wn, sage2_b.reshape(1, -1))

    f = pl.pallas_call(
        _img_body,
        out_shape=jax.ShapeDtypeStruct((img.shape[0], 800), jnp.float32),
    )(img, img_w1, img_b1.reshape(1, -1), img_g1.reshape(1, -1),
      img_be1.reshape(1, -1), img_w2, img_b2.reshape(1, -1),
      img_g2.reshape(1, -1), img_be2.reshape(1, -1), img_w3,
      img_b3.reshape(1, -1), img_go.reshape(1, -1), img_beo.reshape(1, -1))

    attr_pad = jnp.zeros((128, 512), jnp.float32).at[:_NATTRS].set(
        h2[:_NATTRS])
    obj_pad = jnp.zeros((256, 512), jnp.float32).at[:_NOBJS].set(
        h2[_NATTRS:])
    wa = pair_w1[:512]
    wb = pair_w1[512:]

    out = pl.pallas_call(
        _pair_body,
        grid=(_PGRID,),
        in_specs=[
            pl.BlockSpec((128, 512), _full),
            pl.BlockSpec((256, 512), _full),
            pl.BlockSpec((512, 1200), _full),
            pl.BlockSpec((512, 1200), _full),
            pl.BlockSpec((1, 1200), _full),
            pl.BlockSpec((1, 1200), _full),
            pl.BlockSpec((1, 1200), _full),
            pl.BlockSpec((1200, 800), _full),
            pl.BlockSpec((1, 800), _full),
            pl.BlockSpec((1, 800), _full),
            pl.BlockSpec((1, 800), _full),
            pl.BlockSpec((1024, 800), _full),
        ],
        out_specs=pl.BlockSpec((_PBLK, 1024), lambda i: (i, 0)),
        out_shape=jax.ShapeDtypeStruct((_NPAIRS, 1024), jnp.float32),
        scratch_shapes=[
            pltpu.VMEM((128, 1200), jnp.float32),
            pltpu.VMEM((256, 1200), jnp.float32),
        ],
    )(attr_pad, obj_pad, wa, wb, pair_b1.reshape(1, -1),
      pair_g1.reshape(1, -1), pair_be1.reshape(1, -1), pair_w2,
      pair_b2.reshape(1, -1), pair_go.reshape(1, -1),
      pair_beo.reshape(1, -1), f)
    return out.T


# speedup vs baseline: 1.4073x; 1.0826x over previous
import functools

import jax
import jax.numpy as jnp
from jax import lax
from jax.experimental import pallas as pl
from jax.experimental.pallas import tpu as pltpu
from jax.experimental.pallas import tpu_sc as plsc

_NATTRS = 115
_NOBJS = 245
_NNODES = _NATTRS + _NOBJS
_NEDGES = 57000

_NC = 2
_NS = 16
_NW = _NC * _NS
_CHUNK = 128
_NCHUNKS = 14
_EPW = _CHUNK * _NCHUNKS
_NE_PAD = _NW * _EPW
_AVALID = _NNODES * _NNODES
_ASIZE = 131072
_SLICE = _ASIZE // _NS

_PBLK = 2048
_NPAIRS = _NATTRS * _NOBJS
_PGRID = (_NPAIRS + _PBLK - 1) // _PBLK


def _sc_adj_body(src_hbm, dst_hbm, out_hbm, src_v, dst_v, idx_v, ones_v,
                 stage_v, shared):
    c = lax.axis_index("c")
    s = lax.axis_index("s")
    wid = s * _NC + c
    base = wid * _EPW

    def _z(i, carry):
        stage_v[pl.ds(i * 16, 16)] = jnp.zeros((16,), jnp.float32)
        return carry

    lax.fori_loop(0, _SLICE // 16, _z, 0)
    pltpu.sync_copy(stage_v, shared.at[pl.ds(s * _SLICE, _SLICE)])

    pltpu.sync_copy(src_hbm.at[pl.ds(base, _EPW)], src_v)
    pltpu.sync_copy(dst_hbm.at[pl.ds(base, _EPW)], dst_v)
    for t in range(_CHUNK // 16):
        ones_v[pl.ds(t * 16, 16)] = jnp.ones((16,), jnp.float32)

    for k in range(_NCHUNKS):
        for t in range(_CHUNK // 16):
            o = k * _CHUNK + t * 16
            d16 = dst_v[pl.ds(o, 16)]
            s16 = src_v[pl.ds(o, 16)]
            idx_v[k, pl.ds(t * 16, 16)] = d16 * _NNODES + s16

    plsc.subcore_barrier()
    for k in range(_NCHUNKS):
        pltpu.sync_copy(ones_v, shared.at[idx_v.at[k]], add=True)
    plsc.subcore_barrier()

    pltpu.sync_copy(shared.at[pl.ds(s * _SLICE, _SLICE)], stage_v)
    pltpu.sync_copy(stage_v, out_hbm.at[c, pl.ds(s * _SLICE, _SLICE)])


def _build_adjacency(src_p, dst_p):
    return pl.kernel(
        _sc_adj_body,
        out_type=jax.ShapeDtypeStruct((_NC, _ASIZE), jnp.float32),
        mesh=plsc.VectorSubcoreMesh(
            core_axis_name="c", subcore_axis_name="s",
            num_cores=_NC, num_subcores=_NS),
        scratch_types=[
            pltpu.VMEM((_EPW,), jnp.int32),
            pltpu.VMEM((_EPW,), jnp.int32),
            pltpu.VMEM((_NCHUNKS, _CHUNK), jnp.int32),
            pltpu.VMEM((_CHUNK,), jnp.float32),
            pltpu.VMEM((_SLICE,), jnp.float32),
            pltpu.VMEM_SHARED((_ASIZE,), jnp.float32),
        ],
    )(src_p, dst_p)


def _ln(x, g, b):
    m = jnp.mean(x, axis=-1, keepdims=True)
    v = jnp.mean((x - m) * (x - m), axis=-1, keepdims=True)
    return (x - m) / jnp.sqrt(v + 1e-5) * g + b


def _dot(a, b):
    return jnp.dot(a, b, preferred_element_type=jnp.float32)


def _gnn_body(a_ref, nodes_ref, wr1, wn1, b1, wr2, wn2, b2, h2_out):
    A = a_ref[0] + a_ref[1]
    deg = jnp.sum(A, axis=1, keepdims=True)
    An = A / jnp.maximum(deg, 1.0)
    x = nodes_ref[...]
    agg1 = _dot(An, x)
    h1 = jnp.maximum(_dot(x, wr1[...]) + _dot(agg1, wn1[...]) + b1[...], 0.0)
    agg2 = _dot(An, h1)
    h2_out[...] = _dot(h1, wr2[...]) + _dot(agg2, wn2[...]) + b2[...]


def _dot_t(a, bt):
    return lax.dot_general(a, bt, (((1,), (1,)), ((), ())),
                           preferred_element_type=jnp.float32)


def _img_body(img_ref, w1, b1, g1, be1, w2t, b2, g2, be2, w3t, b3, go, beo,
              f_out):
    x = img_ref[...]
    x = jnp.maximum(_ln(_dot(x, w1[...]) + b1[...], g1[...], be1[...]), 0.0)
    x = jnp.maximum(_ln(_dot_t(x, w2t[...]) + b2[...], g2[...], be2[...]),
                    0.0)
    f_out[...] = _ln(_dot_t(x, w3t[...]) + b3[...], go[...], beo[...])


def _pair_body(attr_ref, obj_ref, w1t, b1, g1, be1, w2t, b2, go, beo,
               f_ref, out_ref, ap_s, op_s):
    i = pl.program_id(0)

    @pl.when(i == 0)
    def _():
        ap_s[...] = _dot_t(attr_ref[...], w1t[:, 0:512])
        op_s[...] = _dot_t(obj_ref[...], w1t[:, 512:1024])

    r = i * _PBLK + lax.broadcasted_iota(jnp.int32, (_PBLK, 1), 0)
    ia = r // _NOBJS
    jo = r - ia * _NOBJS
    aoh = (ia == lax.broadcasted_iota(jnp.int32, (_PBLK, 128), 1)
           ).astype(jnp.float32)
    boh = (jo == lax.broadcasted_iota(jnp.int32, (_PBLK, 256), 1)
           ).astype(jnp.float32)
    x = _dot(aoh, ap_s[...]) + _dot(boh, op_s[...]) + b1[...]
    x = jnp.maximum(_ln(x, g1[...], be1[...]), 0.0)
    q = _dot_t(x, w2t[...]) + b2[...]
    p = _ln(q, go[...], beo[...])
    out_ref[...] = lax.dot_general(
        p, f_ref[...], (((1,), (1,)), ((), ())),
        preferred_element_type=jnp.float32)


def _full(i):
    return (0, 0)


def kernel(img, edge_index, nodes, sage1_wr, sage1_wn, sage1_b, sage2_wr,
           sage2_wn, sage2_b, img_w1, img_b1, img_g1, img_be1, img_w2,
           img_b2, img_g2, img_be2, img_w3, img_b3, img_go, img_beo,
           pair_w1, pair_b1, pair_g1, pair_be1, pair_w2, pair_b2, pair_go,
           pair_beo):
    src = edge_index[0].astype(jnp.int32)
    dst = edge_index[1].astype(jnp.int32)
    pad = _NE_PAD - _NEDGES
    src_p = jnp.concatenate([src, jnp.zeros((pad,), jnp.int32)])
    dst_p = jnp.concatenate([dst, jnp.full((pad,), _NNODES, jnp.int32)])

    a2 = _build_adjacency(src_p, dst_p)
    a = a2[:, :_AVALID].reshape(_NC, _NNODES, _NNODES)

    h2 = pl.pallas_call(
        _gnn_body,
        out_shape=jax.ShapeDtypeStruct((_NNODES, 512), jnp.float32),
    )(a, nodes, sage1_wr, sage1_wn, sage1_b.reshape(1, -1),
      sage2_wr, sage2_wn, sage2_b.reshape(1, -1))

    f = pl.pallas_call(
        _img_body,
        out_shape=jax.ShapeDtypeStruct((img.shape[0], 800), jnp.float32),
    )(img, img_w1, img_b1.reshape(1, -1), img_g1.reshape(1, -1),
      img_be1.reshape(1, -1), img_w2.T, img_b2.reshape(1, -1),
      img_g2.reshape(1, -1), img_be2.reshape(1, -1), img_w3.T,
      img_b3.reshape(1, -1), img_go.reshape(1, -1), img_beo.reshape(1, -1))

    attr_pad = jnp.zeros((128, 512), jnp.float32).at[:_NATTRS].set(
        h2[:_NATTRS])
    obj_pad = jnp.zeros((256, 512), jnp.float32).at[:_NOBJS].set(
        h2[_NATTRS:])

    out = pl.pallas_call(
        _pair_body,
        grid=(_PGRID,),
        in_specs=[
            pl.BlockSpec((128, 512), _full),
            pl.BlockSpec((256, 512), _full),
            pl.BlockSpec((1200, 1024), _full),
            pl.BlockSpec((1, 1200), _full),
            pl.BlockSpec((1, 1200), _full),
            pl.BlockSpec((1, 1200), _full),
            pl.BlockSpec((800, 1200), _full),
            pl.BlockSpec((1, 800), _full),
            pl.BlockSpec((1, 800), _full),
            pl.BlockSpec((1, 800), _full),
            pl.BlockSpec((1024, 800), _full),
        ],
        out_specs=pl.BlockSpec((_PBLK, 1024), lambda i: (i, 0)),
        out_shape=jax.ShapeDtypeStruct((_NPAIRS, 1024), jnp.float32),
        scratch_shapes=[
            pltpu.VMEM((128, 1200), jnp.float32),
            pltpu.VMEM((256, 1200), jnp.float32),
        ],
    )(attr_pad, obj_pad, pair_w1.T, pair_b1.reshape(1, -1),
      pair_g1.reshape(1, -1), pair_be1.reshape(1, -1), pair_w2.T,
      pair_b2.reshape(1, -1), pair_go.reshape(1, -1),
      pair_beo.reshape(1, -1), f)
    return out.T
